# baseline (device time: 9142 ns/iter reference)
import jax
import jax.numpy as jnp
from jax import lax
from jax.experimental import pallas as pl
from jax.experimental.pallas import tpu as pltpu


def kernel(x, dy, gamma):
    m, d = x.shape
    half = m // 2

    off = lax.axis_index("y") * half
    stacked = jnp.stack([
        lax.dynamic_slice(x, (off, 0), (half, d)),
        lax.dynamic_slice(dy, (off, 0), (half, d)),
    ])

    def body(s_ref, out_ref, comm_ref, send_sems, recv_sems):
        my_x = lax.axis_index("x")
        my_y = lax.axis_index("y")
        peers = (
            (1 - my_x, my_y),
            (my_x, 1 - my_y),
            (1 - my_x, 1 - my_y),
        )

        barrier_sem = pltpu.get_barrier_semaphore()
        for nbr in peers:
            pl.semaphore_signal(
                barrier_sem, inc=1,
                device_id=nbr, device_id_type=pl.DeviceIdType.MESH,
            )

        xv = s_ref[0]
        dyv = s_ref[1]
        inv_d = jnp.float32(1.0 / d)
        mu = jnp.sum(xv, axis=1, keepdims=True) * inv_d
        mean2 = jnp.sum(xv * xv, axis=1, keepdims=True) * inv_d
        rstd = lax.rsqrt(mean2 - mu * mu + 1e-5)
        comm_ref[0, 0, :] = jnp.sum(dyv * ((xv - mu) * rstd), axis=0)
        comm_ref[0, 1, :] = jnp.sum(dyv, axis=0)

        pl.semaphore_wait(barrier_sem, 3)

        rdmas = []
        for i, nbr in enumerate(peers):
            rdma = pltpu.make_async_remote_copy(
                src_ref=comm_ref.at[0], dst_ref=comm_ref.at[i + 1],
                send_sem=send_sems.at[i], recv_sem=recv_sems.at[i],
                device_id=nbr, device_id_type=pl.DeviceIdType.MESH,
            )
            rdma.start()
            rdmas.append(rdma)
        for rdma in rdmas:
            rdma.wait_recv()
        out_ref[:, :] = (comm_ref[0] + comm_ref[1]) + (comm_ref[2] + comm_ref[3])
        for rdma in rdmas:
            rdma.wait_send()

    return pl.pallas_call(
        body,
        out_shape=jax.ShapeDtypeStruct((2, d), jnp.float32),
        in_specs=[pl.BlockSpec(memory_space=pltpu.VMEM)],
        out_specs=pl.BlockSpec(memory_space=pltpu.VMEM),
        scratch_shapes=[
            pltpu.VMEM((4, 2, d), jnp.float32),
            pltpu.SemaphoreType.DMA((3,)),
            pltpu.SemaphoreType.DMA((3,)),
        ],
        compiler_params=pltpu.CompilerParams(collective_id=0),
    )(stacked)
